# speedup vs baseline: 23.6563x; 23.6563x over previous
import functools

import jax
import jax.numpy as jnp
import numpy as np
from jax.experimental import pallas as pl
from jax.experimental.pallas import tpu as pltpu

_MASK_ID = 99999
_PAD_ID = 0
_MAX_STEPS = 64
_DT = (1.0 - 1e-05) / (_MAX_STEPS + 1)

_KS0 = np.uint32(1948878966)
_KS1 = np.uint32(4237131848)
_KS2 = np.uint32(int(_KS0) ^ int(_KS1) ^ 0x1BD11BDA)
_KS = (_KS0, _KS1, _KS2)
_ROT = ((13, 15, 26, 6), (17, 29, 16, 24))

_ROWS = 8


def _threefry_bits(n):
    x0 = jnp.full(n.shape, _KS0, dtype=jnp.uint32)
    x1 = n + _KS1
    for step in range(5):
        for r in _ROT[step % 2]:
            x0 = x0 + x1
            x1 = ((x1 << r) | (x1 >> (32 - r))) ^ x0
        x0 = x0 + _KS[(step + 1) % 3]
        x1 = x1 + _KS[(step + 2) % 3] + np.uint32(step + 1)
    return x0 ^ x1


def _tau_step_kernel(coef_ref, xt_ref, vl_ref, rate_ref, newxt_ref, *, V):
    i = pl.program_id(0)
    x = vl_ref[0]
    coef = coef_ref[0]
    xtv = xt_ref[0]

    lane = jax.lax.broadcasted_iota(jnp.int32, (_ROWS, V), 1)

    rowmax = jnp.max(x, axis=1, keepdims=True)
    e = jnp.exp(x - rowmax)
    s = jnp.sum(e, axis=1, keepdims=True)
    rate = e * (coef / s)
    rate = jnp.where(lane == (V - 1), jnp.float32(0.0), rate)
    rate_ref[0] = rate

    row = jax.lax.broadcasted_iota(jnp.int32, (_ROWS, V), 0) + i * _ROWS
    n = (row * V + lane).astype(jnp.uint32)
    bits = _threefry_bits(n)
    u = jax.lax.bitcast_convert_type(
        (bits >> 9) | np.uint32(0x3F800000), jnp.float32) - jnp.float32(1.0)
    logu = jnp.log(jnp.maximum(u, jnp.float32(1e-37)))
    cond = (rate > 0) & (logu > -rate)
    ev = jnp.max(cond.astype(jnp.int32), axis=1, keepdims=True) > 0

    amax = jnp.min(jnp.where(x == rowmax, lane, V), axis=1, keepdims=True)
    is_masked = xtv == _MASK_ID
    new = jnp.where(is_masked & ev, amax, xtv)
    new = jnp.where(xtv == _PAD_ID, _PAD_ID, new)
    newxt_ref[0] = new


@jax.jit
def kernel(vocab_logits, xt, t):
    B, L, V = vocab_logits.shape
    n_rows = B * L
    n_blocks = n_rows // _ROWS

    vl4 = vocab_logits.reshape(n_blocks, _ROWS, V)
    xt3 = xt.reshape(n_blocks, _ROWS, 1)
    factor = 1.0 / (1.0 - t + 1e-4)
    coef = jnp.float32(_DT) * factor[:, None] * (xt == _MASK_ID)
    coef3 = coef.astype(jnp.float32).reshape(n_blocks, _ROWS, 1)

    rate4, newxt3 = pl.pallas_call(
        functools.partial(_tau_step_kernel, V=V),
        grid=(n_blocks,),
        in_specs=[
            pl.BlockSpec((1, _ROWS, 1), lambda i: (i, 0, 0)),
            pl.BlockSpec((1, _ROWS, 1), lambda i: (i, 0, 0)),
            pl.BlockSpec((1, _ROWS, V), lambda i: (i, 0, 0)),
        ],
        out_specs=[
            pl.BlockSpec((1, _ROWS, V), lambda i: (i, 0, 0)),
            pl.BlockSpec((1, _ROWS, 1), lambda i: (i, 0, 0)),
        ],
        out_shape=[
            jax.ShapeDtypeStruct((n_blocks, _ROWS, V), jnp.float32),
            jax.ShapeDtypeStruct((n_blocks, _ROWS, 1), jnp.int32),
        ],
        compiler_params=pltpu.CompilerParams(
            dimension_semantics=("parallel",),
        ),
    )(coef3, xt3, vl4)

    return newxt3.reshape(B, L), rate4.reshape(B, L, V)


# speedup vs baseline: 24.1859x; 1.0224x over previous
import functools

import jax
import jax.numpy as jnp
import numpy as np
from jax.experimental import pallas as pl
from jax.experimental.pallas import tpu as pltpu

_MASK_ID = 99999
_PAD_ID = 0
_MAX_STEPS = 64
_DT = (1.0 - 1e-05) / (_MAX_STEPS + 1)

_KS0 = np.uint32(1948878966)
_KS1 = np.uint32(4237131848)
_KS2 = np.uint32(int(_KS0) ^ int(_KS1) ^ 0x1BD11BDA)
_KS = (_KS0, _KS1, _KS2)
_ROT = ((13, 15, 26, 6), (17, 29, 16, 24))

_ROWS = 8


def _threefry_bits(n):
    x0 = jnp.full(n.shape, _KS0, dtype=jnp.uint32)
    x1 = n + _KS1
    for step in range(5):
        for r in _ROT[step % 2]:
            x0 = x0 + x1
            x1 = ((x1 << r) | (x1 >> (32 - r))) ^ x0
        x0 = x0 + _KS[(step + 1) % 3]
        x1 = x1 + _KS[(step + 2) % 3] + np.uint32(step + 1)
    return x0 ^ x1


def _tau_step_kernel(coef_ref, xt_ref, vl_ref, rate_ref, newxt_ref, *, V):
    i = pl.program_id(0)
    x = vl_ref[0]
    coef = coef_ref[0]
    xtv = xt_ref[0]

    lane = jax.lax.broadcasted_iota(jnp.int32, (_ROWS, V), 1)

    rowmax = jnp.max(x, axis=1, keepdims=True)
    e = jnp.exp(x - rowmax)
    s = jnp.sum(e, axis=1, keepdims=True)
    rate = e * (coef / s)
    rate = jnp.where(lane == (V - 1), jnp.float32(0.0), rate)
    rate_ref[0] = rate

    row = jax.lax.broadcasted_iota(jnp.int32, (_ROWS, V), 0) + i * _ROWS
    n = (row * V + lane).astype(jnp.uint32)
    bits = _threefry_bits(n)
    u = jax.lax.bitcast_convert_type(
        (bits >> 9) | np.uint32(0x3F800000), jnp.float32) - jnp.float32(1.0)
    cond = jnp.log(u) > -rate
    ev = jnp.max(cond.astype(jnp.int32), axis=1, keepdims=True) > 0

    new = jnp.where(xtv == _PAD_ID, _PAD_ID, xtv)
    newxt_ref[0] = new

    @pl.when(jnp.max(ev.astype(jnp.int32)) > 0)
    def _reveal():
        amax = jnp.min(jnp.where(x == rowmax, lane, V), axis=1, keepdims=True)
        is_masked = xtv == _MASK_ID
        new2 = jnp.where(is_masked & ev, amax, new)
        newxt_ref[0] = jnp.where(xtv == _PAD_ID, _PAD_ID, new2)


@jax.jit
def kernel(vocab_logits, xt, t):
    B, L, V = vocab_logits.shape
    n_rows = B * L
    n_blocks = n_rows // _ROWS

    vl4 = vocab_logits.reshape(n_blocks, _ROWS, V)
    xt3 = xt.reshape(n_blocks, _ROWS, 1)
    factor = 1.0 / (1.0 - t + 1e-4)
    coef = jnp.float32(_DT) * factor[:, None] * (xt == _MASK_ID)
    coef3 = coef.astype(jnp.float32).reshape(n_blocks, _ROWS, 1)

    rate4, newxt3 = pl.pallas_call(
        functools.partial(_tau_step_kernel, V=V),
        grid=(n_blocks,),
        in_specs=[
            pl.BlockSpec((1, _ROWS, 1), lambda i: (i, 0, 0)),
            pl.BlockSpec((1, _ROWS, 1), lambda i: (i, 0, 0)),
            pl.BlockSpec((1, _ROWS, V), lambda i: (i, 0, 0)),
        ],
        out_specs=[
            pl.BlockSpec((1, _ROWS, V), lambda i: (i, 0, 0)),
            pl.BlockSpec((1, _ROWS, 1), lambda i: (i, 0, 0)),
        ],
        out_shape=[
            jax.ShapeDtypeStruct((n_blocks, _ROWS, V), jnp.float32),
            jax.ShapeDtypeStruct((n_blocks, _ROWS, 1), jnp.int32),
        ],
        compiler_params=pltpu.CompilerParams(
            dimension_semantics=("parallel",),
        ),
    )(coef3, xt3, vl4)

    return newxt3.reshape(B, L), rate4.reshape(B, L, V)


# speedup vs baseline: 24.1878x; 1.0001x over previous
import functools

import jax
import jax.numpy as jnp
import numpy as np
from jax.experimental import pallas as pl
from jax.experimental.pallas import tpu as pltpu

_MASK_ID = 99999
_PAD_ID = 0
_MAX_STEPS = 64
_DT = (1.0 - 1e-05) / (_MAX_STEPS + 1)

_KS0 = np.uint32(1948878966)
_KS1 = np.uint32(4237131848)
_KS2 = np.uint32(int(_KS0) ^ int(_KS1) ^ 0x1BD11BDA)
_KS = (_KS0, _KS1, _KS2)
_ROT = ((13, 15, 26, 6), (17, 29, 16, 24))

_ROWS = 8


def _threefry_bits(n):
    x0 = jnp.full(n.shape, _KS0, dtype=jnp.uint32)
    x1 = n + _KS1
    for step in range(5):
        for r in _ROT[step % 2]:
            x0 = x0 + x1
            x1 = ((x1 << r) | (x1 >> (32 - r))) ^ x0
        x0 = x0 + _KS[(step + 1) % 3]
        x1 = x1 + _KS[(step + 2) % 3] + np.uint32(step + 1)
    return x0 ^ x1


def _tau_step_kernel(coef_ref, xt_ref, vl_ref, rate_ref, newxt_ref, *, V):
    i = pl.program_id(0)
    x = vl_ref[0]
    coef = coef_ref[0]
    xtv = xt_ref[0]

    lane = jax.lax.broadcasted_iota(jnp.int32, (_ROWS, V), 1)

    rowmax = jnp.max(x, axis=1, keepdims=True)
    e = jnp.exp(x - rowmax)
    s = jnp.sum(e, axis=1, keepdims=True)
    rate = e * (coef / s)
    rate = jnp.where(lane == (V - 1), jnp.float32(0.0), rate)
    rate_ref[0] = rate

    row = jax.lax.broadcasted_iota(jnp.int32, (_ROWS, V), 0) + i * _ROWS
    n = (row * V + lane).astype(jnp.uint32)
    bits = _threefry_bits(n)
    u = jax.lax.bitcast_convert_type(
        (bits >> 9) | np.uint32(0x3F800000), jnp.float32) - jnp.float32(1.0)
    cond = jnp.log(u) > -rate
    ev = jnp.max(cond.astype(jnp.int32), axis=1, keepdims=True) > 0

    new = jnp.where(xtv == _PAD_ID, _PAD_ID, xtv)
    newxt_ref[0] = new

    @pl.when(jnp.max(ev.astype(jnp.int32)) > 0)
    def _reveal():
        amax = jnp.min(jnp.where(x == rowmax, lane, V), axis=1, keepdims=True)
        is_masked = xtv == _MASK_ID
        new2 = jnp.where(is_masked & ev, amax, new)
        newxt_ref[0] = jnp.where(xtv == _PAD_ID, _PAD_ID, new2)


@jax.jit
def kernel(vocab_logits, xt, t):
    B, L, V = vocab_logits.shape
    n_rows = B * L
    n_blocks = n_rows // _ROWS

    vl4 = vocab_logits.reshape(n_blocks, _ROWS, V)
    xt3 = xt.reshape(n_blocks, _ROWS, 1)
    factor = 1.0 / (1.0 - t + 1e-4)
    coef = jnp.float32(_DT) * factor[:, None] * (xt == _MASK_ID)
    coef3 = coef.astype(jnp.float32).reshape(n_blocks, _ROWS, 1)

    rate4, newxt3 = pl.pallas_call(
        functools.partial(_tau_step_kernel, V=V),
        grid=(n_blocks,),
        in_specs=[
            pl.BlockSpec((1, _ROWS, 1), lambda i: (i, 0, 0)),
            pl.BlockSpec((1, _ROWS, 1), lambda i: (i, 0, 0)),
            pl.BlockSpec((1, _ROWS, V), lambda i: (i, 0, 0)),
        ],
        out_specs=[
            pl.BlockSpec((1, _ROWS, V), lambda i: (i, 0, 0)),
            pl.BlockSpec((1, _ROWS, 1), lambda i: (i, 0, 0)),
        ],
        out_shape=[
            jax.ShapeDtypeStruct((n_blocks, _ROWS, V), jnp.float32),
            jax.ShapeDtypeStruct((n_blocks, _ROWS, 1), jnp.int32),
        ],
        compiler_params=pltpu.CompilerParams(
            dimension_semantics=("arbitrary",),
        ),
    )(coef3, xt3, vl4)

    return newxt3.reshape(B, L), rate4.reshape(B, L, V)


# speedup vs baseline: 35.0824x; 1.4504x over previous
import functools

import jax
import jax.numpy as jnp
import numpy as np
from jax.experimental import pallas as pl
from jax.experimental.pallas import tpu as pltpu

_MASK_ID = 99999
_PAD_ID = 0
_MAX_STEPS = 64
_DT = (1.0 - 1e-05) / (_MAX_STEPS + 1)

_KS0 = np.uint32(1948878966)
_KS1 = np.uint32(4237131848)
_KS2 = np.uint32(int(_KS0) ^ int(_KS1) ^ 0x1BD11BDA)
_KS = (_KS0, _KS1, _KS2)
_ROT = ((13, 15, 26, 6), (17, 29, 16, 24))

_ROWS = 8
_CH = 1024


def _threefry_bits(n):
    x0 = jnp.full(n.shape, _KS0, dtype=jnp.uint32)
    x1 = n + _KS1
    for step in range(5):
        for r in _ROT[step % 2]:
            x0 = x0 + x1
            x1 = ((x1 << r) | (x1 >> (32 - r))) ^ x0
        x0 = x0 + _KS[(step + 1) % 3]
        x1 = x1 + _KS[(step + 2) % 3] + np.uint32(step + 1)
    return x0 ^ x1


def _tau_step_kernel(coef_ref, xt_ref, vl_ref, rate_ref, newxt_ref, *, V):
    i = pl.program_id(0)
    coef = coef_ref[0]
    xtv = xt_ref[0]
    n_chunks = (V + _CH - 1) // _CH

    def lanes(start):
        return jax.lax.broadcasted_iota(jnp.int32, (_ROWS, _CH), 1) + start

    def max_body(c, m):
        x_c = vl_ref[0, :, pl.ds(c * _CH, _CH)]
        return jnp.maximum(m, jnp.where(lanes(c * _CH) < V, x_c, -jnp.inf))

    m = jax.lax.fori_loop(
        0, n_chunks, max_body,
        jnp.full((_ROWS, _CH), -jnp.inf, jnp.float32))
    rowmax = jnp.max(m, axis=1, keepdims=True)

    def sum_body(c, s):
        x_c = vl_ref[0, :, pl.ds(c * _CH, _CH)]
        e_c = jnp.exp(x_c - rowmax)
        return s + jnp.where(lanes(c * _CH) < V, e_c, 0.0)

    s = jax.lax.fori_loop(
        0, n_chunks, sum_body, jnp.zeros((_ROWS, _CH), jnp.float32))
    recip = coef / jnp.sum(s, axis=1, keepdims=True)

    subl = jax.lax.broadcasted_iota(jnp.int32, (_ROWS, 1), 0)
    rowV = (i * _ROWS + subl) * V

    def b_body(c, acc):
        start = c * _CH
        gidx = lanes(start)
        x_c = vl_ref[0, :, pl.ds(start, _CH)]
        e_c = jnp.exp(x_c - rowmax)
        r_c = e_c * recip
        r_c = jnp.where(gidx == (V - 1), jnp.float32(0.0), r_c)
        rate_ref[0, :, pl.ds(start, _CH)] = r_c
        n = (rowV + gidx).astype(jnp.uint32)
        bits = _threefry_bits(n)
        u = jax.lax.bitcast_convert_type(
            (bits >> 9) | np.uint32(0x3F800000), jnp.float32) - jnp.float32(1.0)
        cond = (jnp.log(u) > -r_c) & (gidx < V)
        return acc | cond.astype(jnp.int32)

    acc = jax.lax.fori_loop(
        0, n_chunks, b_body, jnp.zeros((_ROWS, _CH), jnp.int32))
    ev = jnp.max(acc, axis=1, keepdims=True) > 0

    newxt_ref[0] = xtv

    @pl.when(jnp.max(ev.astype(jnp.int32)) > 0)
    def _reveal():
        def argmax_body(c, best):
            gidx = lanes(c * _CH)
            x_c = vl_ref[0, :, pl.ds(c * _CH, _CH)]
            hit = (x_c == rowmax) & (gidx < V)
            return jnp.minimum(best, jnp.where(hit, gidx, V))

        best = jax.lax.fori_loop(
            0, n_chunks, argmax_body, jnp.full((_ROWS, _CH), V, jnp.int32))
        amax = jnp.min(best, axis=1, keepdims=True)
        is_masked = xtv == _MASK_ID
        new = jnp.where(is_masked & ev, amax, xtv)
        newxt_ref[0] = jnp.where(xtv == _PAD_ID, _PAD_ID, new)


@jax.jit
def kernel(vocab_logits, xt, t):
    B, L, V = vocab_logits.shape
    n_rows = B * L
    n_blocks = n_rows // _ROWS
    VP = ((V + _CH - 1) // _CH) * _CH

    vl4 = vocab_logits.reshape(n_blocks, _ROWS, V)
    xt3 = xt.reshape(n_blocks, _ROWS, 1)
    factor = 1.0 / (1.0 - t + 1e-4)
    coef = jnp.float32(_DT) * factor[:, None] * (xt == _MASK_ID)
    coef3 = coef.astype(jnp.float32).reshape(n_blocks, _ROWS, 1)

    rate4, newxt3 = pl.pallas_call(
        functools.partial(_tau_step_kernel, V=V),
        grid=(n_blocks,),
        in_specs=[
            pl.BlockSpec((1, _ROWS, 1), lambda i: (i, 0, 0)),
            pl.BlockSpec((1, _ROWS, 1), lambda i: (i, 0, 0)),
            pl.BlockSpec((1, _ROWS, VP), lambda i: (i, 0, 0)),
        ],
        out_specs=[
            pl.BlockSpec((1, _ROWS, VP), lambda i: (i, 0, 0)),
            pl.BlockSpec((1, _ROWS, 1), lambda i: (i, 0, 0)),
        ],
        out_shape=[
            jax.ShapeDtypeStruct((n_blocks, _ROWS, V), jnp.float32),
            jax.ShapeDtypeStruct((n_blocks, _ROWS, 1), jnp.int32),
        ],
        compiler_params=pltpu.CompilerParams(
            dimension_semantics=("arbitrary",),
        ),
    )(coef3, xt3, vl4)

    return newxt3.reshape(B, L), rate4.reshape(B, L, V)


# speedup vs baseline: 35.3481x; 1.0076x over previous
import functools

import jax
import jax.numpy as jnp
import numpy as np
from jax.experimental import pallas as pl
from jax.experimental.pallas import tpu as pltpu

_MASK_ID = 99999
_PAD_ID = 0
_MAX_STEPS = 64
_DT = (1.0 - 1e-05) / (_MAX_STEPS + 1)

_KS0 = np.uint32(1948878966)
_KS1 = np.uint32(4237131848)
_KS2 = np.uint32(int(_KS0) ^ int(_KS1) ^ 0x1BD11BDA)
_KS = (_KS0, _KS1, _KS2)
_ROT = ((13, 15, 26, 6), (17, 29, 16, 24))

_ROWS = 8
_CH = 2048


def _threefry_bits(n):
    x0 = jnp.full(n.shape, _KS0, dtype=jnp.uint32)
    x1 = n + _KS1
    for step in range(5):
        for r in _ROT[step % 2]:
            x0 = x0 + x1
            x1 = ((x1 << r) | (x1 >> (32 - r))) ^ x0
        x0 = x0 + _KS[(step + 1) % 3]
        x1 = x1 + _KS[(step + 2) % 3] + np.uint32(step + 1)
    return x0 ^ x1


def _tau_step_kernel(coef_ref, xt_ref, vl_ref, rate_ref, newxt_ref, *, V):
    i = pl.program_id(0)
    coef = coef_ref[0]
    xtv = xt_ref[0]
    n_chunks = (V + _CH - 1) // _CH

    def lanes(start):
        return jax.lax.broadcasted_iota(jnp.int32, (_ROWS, _CH), 1) + start

    def max_body(c, m):
        x_c = vl_ref[0, :, pl.ds(c * _CH, _CH)]
        return jnp.maximum(m, jnp.where(lanes(c * _CH) < V, x_c, -jnp.inf))

    m = jax.lax.fori_loop(
        0, n_chunks, max_body,
        jnp.full((_ROWS, _CH), -jnp.inf, jnp.float32))
    rowmax = jnp.max(m, axis=1, keepdims=True)

    def sum_body(c, s):
        x_c = vl_ref[0, :, pl.ds(c * _CH, _CH)]
        e_c = jnp.exp(x_c - rowmax)
        return s + jnp.where(lanes(c * _CH) < V, e_c, 0.0)

    s = jax.lax.fori_loop(
        0, n_chunks, sum_body, jnp.zeros((_ROWS, _CH), jnp.float32))
    recip = coef / jnp.sum(s, axis=1, keepdims=True)

    subl = jax.lax.broadcasted_iota(jnp.int32, (_ROWS, 1), 0)
    rowV = (i * _ROWS + subl) * V

    def b_body(c, acc):
        start = c * _CH
        gidx = lanes(start)
        x_c = vl_ref[0, :, pl.ds(start, _CH)]
        e_c = jnp.exp(x_c - rowmax)
        r_c = e_c * recip
        r_c = jnp.where(gidx == (V - 1), jnp.float32(0.0), r_c)
        rate_ref[0, :, pl.ds(start, _CH)] = r_c
        n = (rowV + gidx).astype(jnp.uint32)
        bits = _threefry_bits(n)
        u = jax.lax.bitcast_convert_type(
            (bits >> 9) | np.uint32(0x3F800000), jnp.float32) - jnp.float32(1.0)
        cond = (jnp.log(u) > -r_c) & (gidx < V)
        return acc | cond.astype(jnp.int32)

    acc = jax.lax.fori_loop(
        0, n_chunks, b_body, jnp.zeros((_ROWS, _CH), jnp.int32))
    ev = jnp.max(acc, axis=1, keepdims=True) > 0

    newxt_ref[0] = xtv

    @pl.when(jnp.max(ev.astype(jnp.int32)) > 0)
    def _reveal():
        def argmax_body(c, best):
            gidx = lanes(c * _CH)
            x_c = vl_ref[0, :, pl.ds(c * _CH, _CH)]
            hit = (x_c == rowmax) & (gidx < V)
            return jnp.minimum(best, jnp.where(hit, gidx, V))

        best = jax.lax.fori_loop(
            0, n_chunks, argmax_body, jnp.full((_ROWS, _CH), V, jnp.int32))
        amax = jnp.min(best, axis=1, keepdims=True)
        is_masked = xtv == _MASK_ID
        new = jnp.where(is_masked & ev, amax, xtv)
        newxt_ref[0] = jnp.where(xtv == _PAD_ID, _PAD_ID, new)


@jax.jit
def kernel(vocab_logits, xt, t):
    B, L, V = vocab_logits.shape
    n_rows = B * L
    n_blocks = n_rows // _ROWS
    VP = ((V + _CH - 1) // _CH) * _CH

    vl4 = vocab_logits.reshape(n_blocks, _ROWS, V)
    xt3 = xt.reshape(n_blocks, _ROWS, 1)
    factor = 1.0 / (1.0 - t + 1e-4)
    coef = jnp.float32(_DT) * factor[:, None] * (xt == _MASK_ID)
    coef3 = coef.astype(jnp.float32).reshape(n_blocks, _ROWS, 1)

    rate4, newxt3 = pl.pallas_call(
        functools.partial(_tau_step_kernel, V=V),
        grid=(n_blocks,),
        in_specs=[
            pl.BlockSpec((1, _ROWS, 1), lambda i: (i, 0, 0)),
            pl.BlockSpec((1, _ROWS, 1), lambda i: (i, 0, 0)),
            pl.BlockSpec((1, _ROWS, VP), lambda i: (i, 0, 0)),
        ],
        out_specs=[
            pl.BlockSpec((1, _ROWS, VP), lambda i: (i, 0, 0)),
            pl.BlockSpec((1, _ROWS, 1), lambda i: (i, 0, 0)),
        ],
        out_shape=[
            jax.ShapeDtypeStruct((n_blocks, _ROWS, V), jnp.float32),
            jax.ShapeDtypeStruct((n_blocks, _ROWS, 1), jnp.int32),
        ],
        compiler_params=pltpu.CompilerParams(
            dimension_semantics=("arbitrary",),
        ),
    )(coef3, xt3, vl4)

    return newxt3.reshape(B, L), rate4.reshape(B, L, V)


# speedup vs baseline: 37.9054x; 1.0723x over previous
import functools

import jax
import jax.numpy as jnp
import numpy as np
from jax.experimental import pallas as pl
from jax.experimental.pallas import tpu as pltpu

_MASK_ID = 99999
_PAD_ID = 0
_MAX_STEPS = 64
_DT = (1.0 - 1e-05) / (_MAX_STEPS + 1)

_KS0 = np.uint32(1948878966)
_KS1 = np.uint32(4237131848)
_KS2 = np.uint32(int(_KS0) ^ int(_KS1) ^ 0x1BD11BDA)
_KS = (_KS0, _KS1, _KS2)
_ROT = ((13, 15, 26, 6), (17, 29, 16, 24))

_ROWS = 8
_CH = 2048


def _threefry_bits(n):
    x0 = jnp.full(n.shape, _KS0, dtype=jnp.uint32)
    x1 = n + _KS1
    for step in range(5):
        for r in _ROT[step % 2]:
            x0 = x0 + x1
            x1 = ((x1 << r) | (x1 >> (32 - r))) ^ x0
        x0 = x0 + _KS[(step + 1) % 3]
        x1 = x1 + _KS[(step + 2) % 3] + np.uint32(step + 1)
    return x0 ^ x1


def _tau_step_kernel(coef_ref, xt_ref, vl_ref, rate_ref, newxt_ref, *, V):
    i = pl.program_id(0)
    coef = coef_ref[0]
    xtv = xt_ref[0]
    n_full = (V + _CH - 1) // _CH - 1
    tail = n_full * _CH
    lane0 = jax.lax.broadcasted_iota(jnp.int32, (_ROWS, _CH), 1)
    tlanes = lane0 + tail
    tvalid = tlanes < V

    def max_body(c, m):
        return jnp.maximum(m, vl_ref[0, :, pl.ds(c * _CH, _CH)])

    m = jax.lax.fori_loop(
        0, n_full, max_body, jnp.full((_ROWS, _CH), -jnp.inf, jnp.float32))
    x_t = vl_ref[0, :, pl.ds(tail, _CH)]
    m = jnp.maximum(m, jnp.where(tvalid, x_t, -jnp.inf))
    rowmax = jnp.max(m, axis=1, keepdims=True)

    def sum_body(c, s):
        x_c = vl_ref[0, :, pl.ds(c * _CH, _CH)]
        return s + jnp.exp(x_c - rowmax)

    s = jax.lax.fori_loop(
        0, n_full, sum_body, jnp.zeros((_ROWS, _CH), jnp.float32))
    s = s + jnp.where(tvalid, jnp.exp(x_t - rowmax), 0.0)
    recip = coef / jnp.sum(s, axis=1, keepdims=True)

    subl = jax.lax.broadcasted_iota(jnp.int32, (_ROWS, 1), 0)
    rowV = (i * _ROWS + subl) * V

    def rng_cond(start, r_c):
        n = (lane0 + (rowV + start)).astype(jnp.uint32)
        bits = _threefry_bits(n)
        u = jax.lax.bitcast_convert_type(
            (bits >> 9) | np.uint32(0x3F800000), jnp.float32) - jnp.float32(1.0)
        return jnp.log(u) > -r_c

    def b_body(c, acc):
        start = c * _CH
        x_c = vl_ref[0, :, pl.ds(start, _CH)]
        r_c = jnp.exp(x_c - rowmax) * recip
        rate_ref[0, :, pl.ds(start, _CH)] = r_c
        return acc | rng_cond(start, r_c).astype(jnp.int32)

    acc = jax.lax.fori_loop(
        0, n_full, b_body, jnp.zeros((_ROWS, _CH), jnp.int32))
    r_t = jnp.exp(x_t - rowmax) * recip
    r_t = jnp.where(tlanes == (V - 1), jnp.float32(0.0), r_t)
    rate_ref[0, :, pl.ds(tail, _CH)] = r_t
    acc = acc | (rng_cond(tail, r_t) & tvalid).astype(jnp.int32)
    ev = jnp.max(acc, axis=1, keepdims=True) > 0

    newxt_ref[0] = xtv

    @pl.when(jnp.max(ev.astype(jnp.int32)) > 0)
    def _reveal():
        def argmax_body(c, best):
            gidx = lane0 + c * _CH
            x_c = vl_ref[0, :, pl.ds(c * _CH, _CH)]
            return jnp.minimum(best, jnp.where(x_c == rowmax, gidx, V))

        best = jax.lax.fori_loop(
            0, n_full, argmax_body, jnp.full((_ROWS, _CH), V, jnp.int32))
        hit_t = (x_t == rowmax) & tvalid
        best = jnp.minimum(best, jnp.where(hit_t, tlanes, V))
        amax = jnp.min(best, axis=1, keepdims=True)
        is_masked = xtv == _MASK_ID
        new = jnp.where(is_masked & ev, amax, xtv)
        newxt_ref[0] = jnp.where(xtv == _PAD_ID, _PAD_ID, new)


@jax.jit
def kernel(vocab_logits, xt, t):
    B, L, V = vocab_logits.shape
    n_rows = B * L
    n_blocks = n_rows // _ROWS
    VP = ((V + _CH - 1) // _CH) * _CH

    vl4 = vocab_logits.reshape(n_blocks, _ROWS, V)
    xt3 = xt.reshape(n_blocks, _ROWS, 1)
    factor = 1.0 / (1.0 - t + 1e-4)
    coef = jnp.float32(_DT) * factor[:, None] * (xt == _MASK_ID)
    coef3 = coef.astype(jnp.float32).reshape(n_blocks, _ROWS, 1)

    rate4, newxt3 = pl.pallas_call(
        functools.partial(_tau_step_kernel, V=V),
        grid=(n_blocks,),
        in_specs=[
            pl.BlockSpec((1, _ROWS, 1), lambda i: (i, 0, 0)),
            pl.BlockSpec((1, _ROWS, 1), lambda i: (i, 0, 0)),
            pl.BlockSpec((1, _ROWS, VP), lambda i: (i, 0, 0)),
        ],
        out_specs=[
            pl.BlockSpec((1, _ROWS, VP), lambda i: (i, 0, 0)),
            pl.BlockSpec((1, _ROWS, 1), lambda i: (i, 0, 0)),
        ],
        out_shape=[
            jax.ShapeDtypeStruct((n_blocks, _ROWS, V), jnp.float32),
            jax.ShapeDtypeStruct((n_blocks, _ROWS, 1), jnp.int32),
        ],
        compiler_params=pltpu.CompilerParams(
            dimension_semantics=("arbitrary",),
        ),
    )(coef3, xt3, vl4)

    return newxt3.reshape(B, L), rate4.reshape(B, L, V)
